# baseline (device time: 35966 ns/iter reference)
import jax
import jax.numpy as jnp
from jax import lax
from jax.experimental import pallas as pl
from jax.experimental.pallas import tpu as pltpu

BM = 1024
NSLOTS = 4
EPS = 1e-5


def kernel(x, gamma):
    m, n = x.shape
    n_global = 2 * n
    nblocks = m // BM
    gamma2d = gamma.reshape(1, n)

    def body(x_ref, g_ref, out_ref, xsave, send_buf, recv_buf):
        g = pl.program_id(0)

        @pl.when(g < nblocks)
        def _():
            slot = g % NSLOTS
            xb = x_ref[...]
            xsave[g % 2] = xb
            partial = jnp.sum(xb * xb, axis=1, keepdims=True)
            send_buf[slot] = partial
            recv_buf[slot] = partial

        @pl.when(g > 0)
        def _():
            h = g - 1
            rslot = h % NSLOTS
            xb = xsave[h % 2]
            total = send_buf[rslot] + recv_buf[rslot]
            inv_rms = lax.rsqrt(total / n_global + EPS)
            out_ref[...] = (xb * g_ref[...] * inv_rms).astype(out_ref.dtype)

    return pl.pallas_call(
        body,
        grid=(nblocks + 1,),
        out_shape=jax.ShapeDtypeStruct((m, n), jnp.bfloat16),
        in_specs=[
            pl.BlockSpec((BM, n), lambda g: (jnp.minimum(g, nblocks - 1), 0)),
            pl.BlockSpec((1, n), lambda g: (0, 0)),
        ],
        out_specs=pl.BlockSpec((BM, n), lambda g: (jnp.maximum(g - 1, 0), 0)),
        scratch_shapes=[
            pltpu.VMEM((2, BM, n), jnp.float32),
            pltpu.VMEM((NSLOTS, BM, 1), jnp.float32),
            pltpu.VMEM((NSLOTS, BM, 1), jnp.float32),
        ],
        compiler_params=pltpu.CompilerParams(
            dimension_semantics=("arbitrary",),
            vmem_limit_bytes=64 * 1024 * 1024,
        ),
    )(x, gamma2d)
